# baseline (speedup 1.0000x reference)
import functools

import jax
import jax.numpy as jnp
from jax import lax
from jax.experimental import pallas as pl
from jax.experimental.pallas import tpu as pltpu
from jax.experimental.pallas import tpu_sc as plsc

N = 10000
D = 128
C = 16
NS = 16
RPT = 640
NP = NS * RPT
CH = 128
E = 320000
U = 8
L = 4
NCH = 160
EPT = NCH * CH
EP = NS * EPT
PAD_NODE = NP - 1
UNR = 8


def _loop(n, body):
    lax.fori_loop(jnp.int32(0), jnp.int32(n), lambda i, c: (body(i), c)[1],
                  None)


def _loop_rows(body):
    def outer(i):
        base = i * jnp.int32(UNR)
        for k in range(UNR):
            body(base + jnp.int32(k))
    _loop(RPT // UNR, outer)


def _rsqrt16(d):
    i = plsc.bitcast(d, jnp.int32)
    i = jnp.int32(0x5F3759DF) - lax.shift_right_logical(i, jnp.int32(1))
    y = plsc.bitcast(i, jnp.float32)
    for _ in range(3):
        y = y * (1.5 - 0.5 * d * y * y)
    return y


def _sc_body(src_h, dst_h, y_h, h2_h, g_h, s_sh, src_t, dst_t,
             bbs, tbuf, disb, gsems, ssems):
    tid = lax.axis_index("s")
    rbase = tid * jnp.int32(RPT)
    rows = pl.ds(rbase, RPT)

    def _drain(sem):
        pltpu.make_async_copy(g_h.at[src_t.at[jnp.int32(0)]],
                              bbs[0], sem).wait()

    def _fire_gather(c, slot):
        pltpu.async_copy(g_h.at[src_t.at[c]], bbs[slot], gsems[slot])

    def _fire_scatter(c, slot, src):
        pltpu.async_copy(src, s_sh.at[dst_t.at[c]], ssems[slot], add=True)

    def _pipe(gather_src):
        hop = gather_src is None
        if hop:
            for x in range(L):
                _fire_gather(jnp.int32(x), x % U)

        def outer(i):
            for j in range(U):
                b = i * jnp.int32(U) + jnp.int32(j)
                sf = (j + L) % U

                @pl.when(jnp.logical_and(b + L < NCH, b + L >= U))
                def _():
                    _drain(ssems[sf])

                if hop:
                    @pl.when(b + L < NCH)
                    def _():
                        _fire_gather(b + L, sf)

                    _drain(gsems[j])
                    _fire_scatter(b, j, bbs[j])
                else:
                    _fire_scatter(b, j, gather_src)

        _loop(NCH // U, outer)
        for j in range(U):
            _drain(ssems[j])

    pltpu.sync_copy(src_h.at[tid], src_t)
    pltpu.sync_copy(dst_h.at[tid], dst_t)

    ones_v = jnp.full((C,), 1.0, jnp.float32)

    def _fill_ones(r):
        tbuf[r] = ones_v

    _loop_rows(_fill_ones)
    pltpu.sync_copy(tbuf, s_sh.at[rows])
    plsc.subcore_barrier()

    _pipe(gather_src=tbuf.at[pl.ds(0, CH)])
    plsc.subcore_barrier()

    pltpu.sync_copy(s_sh.at[rows], disb)
    pltpu.sync_copy(y_h.at[rows], tbuf)

    def _dis_row(r):
        dv = _rsqrt16(disb[r])
        disb[r] = dv
        tbuf[r] = tbuf[r] * dv

    _loop_rows(_dis_row)
    d1 = pltpu.async_copy(tbuf, g_h.at[rows], gsems[0])
    d2 = pltpu.async_copy(tbuf, s_sh.at[rows], gsems[1])
    d1.wait()
    d2.wait()
    plsc.subcore_barrier()

    _pipe(gather_src=None)
    plsc.subcore_barrier()

    pltpu.sync_copy(s_sh.at[rows], tbuf)

    def _g1_row(r):
        dv = disb[r]
        tbuf[r] = tbuf[r] * dv * dv

    _loop_rows(_g1_row)
    d1 = pltpu.async_copy(tbuf, g_h.at[rows], gsems[0])
    d2 = pltpu.async_copy(tbuf, s_sh.at[rows], gsems[1])
    d1.wait()
    d2.wait()
    plsc.subcore_barrier()

    _pipe(gather_src=None)
    plsc.subcore_barrier()

    pltpu.sync_copy(s_sh.at[rows], tbuf)

    def _h2_row(r):
        tbuf[r] = tbuf[r] * disb[r]

    _loop_rows(_h2_row)
    pltpu.sync_copy(tbuf, h2_h.at[rows])


_sc_prop = functools.partial(
    pl.kernel,
    out_type=[
        jax.ShapeDtypeStruct((NP, C), jnp.float32),
        jax.ShapeDtypeStruct((NP, C), jnp.float32),
    ],
    mesh=plsc.VectorSubcoreMesh(
        core_axis_name="c", subcore_axis_name="s", num_cores=1),
    compiler_params=pltpu.CompilerParams(
        needs_layout_passes=False, use_tc_tiling_on_sc=False),
    scratch_types=[
        pltpu.VMEM_SHARED((NP, C), jnp.float32),
        pltpu.VMEM((NCH, CH), jnp.int32),
        pltpu.VMEM((NCH, CH), jnp.int32),
        [pltpu.VMEM((CH, C), jnp.float32) for _ in range(U)],
        pltpu.VMEM((RPT, C), jnp.float32),
        pltpu.VMEM((RPT, C), jnp.float32),
        [pltpu.SemaphoreType.DMA for _ in range(U)],
        [pltpu.SemaphoreType.DMA for _ in range(U)],
    ],
)(_sc_body)


def _matmul_body(x_ref, w_ref, o_ref):
    o_ref[...] = jnp.dot(x_ref[...], w_ref[...],
                         preferred_element_type=jnp.float32)


def _lsm_body(h_ref, b_ref, o_ref):
    t = h_ref[...] + b_ref[...]
    m = jnp.max(t, axis=1, keepdims=True)
    e = jnp.exp(t - m)
    s = jnp.sum(e, axis=1, keepdims=True)
    o_ref[...] = t - m - jnp.log(s)


def kernel(x, edge_index, W, b):
    out_dtype = jnp.result_type(x.dtype, W.dtype, b.dtype)
    x = x.astype(jnp.float32)
    W = W.astype(jnp.float32)
    b = b.astype(jnp.float32)
    ei = jnp.pad(edge_index, ((0, 0), (0, EP - edge_index.shape[1])),
                 constant_values=PAD_NODE).astype(jnp.int32)
    ei = ei.reshape(2, NS, NCH, CH)
    src = ei[0]
    dst = ei[1]
    xp = jnp.pad(x, ((0, NP - N), (0, 0)))

    y = pl.pallas_call(
        _matmul_body,
        out_shape=jax.ShapeDtypeStruct((NP, C), jnp.float32),
    )(xp, W)

    h2, _ = _sc_prop(src, dst, y)

    out = pl.pallas_call(
        _lsm_body,
        out_shape=jax.ShapeDtypeStruct((NP, C), jnp.float32),
    )(h2, b.reshape(1, C))

    return out[:N].astype(out_dtype)


# speedup vs baseline: 1.0154x; 1.0154x over previous
import functools

import jax
import jax.numpy as jnp
from jax import lax
from jax.experimental import pallas as pl
from jax.experimental.pallas import tpu as pltpu
from jax.experimental.pallas import tpu_sc as plsc

N = 10000
D = 128
C = 16
NS = 16
RPT = 640
NP = NS * RPT
CH = 128
E = 320000
U = 10
L = 5
NCH = 160
EPT = NCH * CH
EP = NS * EPT
PAD_NODE = NP - 1
UNR = 8


def _loop(n, body):
    lax.fori_loop(jnp.int32(0), jnp.int32(n), lambda i, c: (body(i), c)[1],
                  None)


def _loop_rows(body):
    def outer(i):
        base = i * jnp.int32(UNR)
        for k in range(UNR):
            body(base + jnp.int32(k))
    _loop(RPT // UNR, outer)


def _rsqrt16(d):
    i = plsc.bitcast(d, jnp.int32)
    i = jnp.int32(0x5F3759DF) - lax.shift_right_logical(i, jnp.int32(1))
    y = plsc.bitcast(i, jnp.float32)
    for _ in range(3):
        y = y * (1.5 - 0.5 * d * y * y)
    return y


def _sc_body(src_h, dst_h, y_h, h2_h, g_h, s_sh, src_t, dst_t,
             bbs, tbuf, disb, gsems, ssems):
    tid = lax.axis_index("s")
    rbase = tid * jnp.int32(RPT)
    rows = pl.ds(rbase, RPT)

    def _drain(sem):
        pltpu.make_async_copy(g_h.at[src_t.at[jnp.int32(0)]],
                              bbs[0], sem).wait()

    def _fire_gather(c, slot):
        pltpu.async_copy(g_h.at[src_t.at[c]], bbs[slot], gsems[slot])

    def _fire_scatter(c, slot, src):
        pltpu.async_copy(src, s_sh.at[dst_t.at[c]], ssems[slot], add=True)

    def _pipe(gather_src):
        hop = gather_src is None
        if hop:
            for x in range(L):
                _fire_gather(jnp.int32(x), x % U)

        def outer(i):
            for j in range(U):
                b = i * jnp.int32(U) + jnp.int32(j)
                sf = (j + L) % U

                @pl.when(jnp.logical_and(b + L < NCH, b + L >= U))
                def _():
                    _drain(ssems[sf])

                if hop:
                    @pl.when(b + L < NCH)
                    def _():
                        _fire_gather(b + L, sf)

                    _drain(gsems[j])
                    _fire_scatter(b, j, bbs[j])
                else:
                    _fire_scatter(b, j, gather_src)

        _loop(NCH // U, outer)
        for j in range(U):
            _drain(ssems[j])

    pltpu.sync_copy(src_h.at[tid], src_t)
    pltpu.sync_copy(dst_h.at[tid], dst_t)

    ones_v = jnp.full((C,), 1.0, jnp.float32)

    def _fill_ones(r):
        tbuf[r] = ones_v

    _loop_rows(_fill_ones)
    pltpu.sync_copy(tbuf, s_sh.at[rows])
    plsc.subcore_barrier()

    _pipe(gather_src=tbuf.at[pl.ds(0, CH)])
    plsc.subcore_barrier()

    pltpu.sync_copy(s_sh.at[rows], disb)
    pltpu.sync_copy(y_h.at[rows], tbuf)

    def _dis_row(r):
        dv = _rsqrt16(disb[r])
        disb[r] = dv
        tbuf[r] = tbuf[r] * dv

    _loop_rows(_dis_row)
    d1 = pltpu.async_copy(tbuf, g_h.at[rows], gsems[0])
    d2 = pltpu.async_copy(tbuf, s_sh.at[rows], gsems[1])
    d1.wait()
    d2.wait()
    plsc.subcore_barrier()

    _pipe(gather_src=None)
    plsc.subcore_barrier()

    pltpu.sync_copy(s_sh.at[rows], tbuf)

    def _g1_row(r):
        dv = disb[r]
        tbuf[r] = tbuf[r] * dv * dv

    _loop_rows(_g1_row)
    d1 = pltpu.async_copy(tbuf, g_h.at[rows], gsems[0])
    d2 = pltpu.async_copy(tbuf, s_sh.at[rows], gsems[1])
    d1.wait()
    d2.wait()
    plsc.subcore_barrier()

    _pipe(gather_src=None)
    plsc.subcore_barrier()

    pltpu.sync_copy(s_sh.at[rows], tbuf)

    def _h2_row(r):
        tbuf[r] = tbuf[r] * disb[r]

    _loop_rows(_h2_row)
    pltpu.sync_copy(tbuf, h2_h.at[rows])


_sc_prop = functools.partial(
    pl.kernel,
    out_type=[
        jax.ShapeDtypeStruct((NP, C), jnp.float32),
        jax.ShapeDtypeStruct((NP, C), jnp.float32),
    ],
    mesh=plsc.VectorSubcoreMesh(
        core_axis_name="c", subcore_axis_name="s", num_cores=1),
    compiler_params=pltpu.CompilerParams(
        needs_layout_passes=False, use_tc_tiling_on_sc=False),
    scratch_types=[
        pltpu.VMEM_SHARED((NP, C), jnp.float32),
        pltpu.VMEM((NCH, CH), jnp.int32),
        pltpu.VMEM((NCH, CH), jnp.int32),
        [pltpu.VMEM((CH, C), jnp.float32) for _ in range(U)],
        pltpu.VMEM((RPT, C), jnp.float32),
        pltpu.VMEM((RPT, C), jnp.float32),
        [pltpu.SemaphoreType.DMA for _ in range(U)],
        [pltpu.SemaphoreType.DMA for _ in range(U)],
    ],
)(_sc_body)


def _matmul_body(x_ref, w_ref, o_ref):
    o_ref[...] = jnp.dot(x_ref[...], w_ref[...],
                         preferred_element_type=jnp.float32)


def _lsm_body(h_ref, b_ref, o_ref):
    t = h_ref[...] + b_ref[...]
    m = jnp.max(t, axis=1, keepdims=True)
    e = jnp.exp(t - m)
    s = jnp.sum(e, axis=1, keepdims=True)
    o_ref[...] = t - m - jnp.log(s)


def kernel(x, edge_index, W, b):
    out_dtype = jnp.result_type(x.dtype, W.dtype, b.dtype)
    x = x.astype(jnp.float32)
    W = W.astype(jnp.float32)
    b = b.astype(jnp.float32)
    ei = jnp.pad(edge_index, ((0, 0), (0, EP - edge_index.shape[1])),
                 constant_values=PAD_NODE).astype(jnp.int32)
    ei = ei.reshape(2, NS, NCH, CH)
    src = ei[0]
    dst = ei[1]
    xp = jnp.pad(x, ((0, NP - N), (0, 0)))

    y = pl.pallas_call(
        _matmul_body,
        out_shape=jax.ShapeDtypeStruct((NP, C), jnp.float32),
    )(xp, W)

    h2, _ = _sc_prop(src, dst, y)

    out = pl.pallas_call(
        _lsm_body,
        out_shape=jax.ShapeDtypeStruct((NP, C), jnp.float32),
    )(h2, b.reshape(1, C))

    return out[:N].astype(out_dtype)


# speedup vs baseline: 1.1208x; 1.1038x over previous
import functools

import jax
import jax.numpy as jnp
from jax import lax
from jax.experimental import pallas as pl
from jax.experimental.pallas import tpu as pltpu
from jax.experimental.pallas import tpu_sc as plsc

N = 10000
D = 128
C = 16
NC = 2
NS = 16
NW = NC * NS
RPW = 320
RPS = 640
NP = NW * RPW
CH = 128
E = 320000
U = 10
L = 5
NCHT = 80
EPT = NCHT * CH
EP = NW * EPT
PAD_NODE = NP - 1
UNR = 8


def _loop(n, body):
    lax.fori_loop(jnp.int32(0), jnp.int32(n), lambda i, c: (body(i), c)[1],
                  None)


def _loop_rows(body):
    def outer(i):
        base = i * jnp.int32(UNR)
        for k in range(UNR):
            body(base + jnp.int32(k))
    _loop(RPW // UNR, outer)


def _rsqrt16(d):
    i = plsc.bitcast(d, jnp.int32)
    i = jnp.int32(0x5F3759DF) - lax.shift_right_logical(i, jnp.int32(1))
    y = plsc.bitcast(i, jnp.float32)
    for _ in range(3):
        y = y * (1.5 - 0.5 * d * y * y)
    return y


def _sc_body(src_h, dst_h, y_h, h2_h, g_h, pub_h, s_sh, src_t, dst_t,
             bbs, tbuf, pbuf, ybuf, disb, zbuf, gsems, ssems, xsem):
    cid = lax.axis_index("c")
    sid = lax.axis_index("s")
    ocid = jnp.int32(1) - cid
    wid = cid * jnp.int32(NS) + sid
    rows32 = pl.ds(wid * jnp.int32(RPW), RPW)
    rows16 = pl.ds(sid * jnp.int32(RPS), RPS)

    def _xbar():
        plsc.subcore_barrier()

        @pl.when(sid == jnp.int32(0))
        def _():
            pl.semaphore_signal(xsem, 1, core_index=ocid)
            pl.semaphore_wait(xsem, 1)

        plsc.subcore_barrier()

    def _drain(sem):
        pltpu.make_async_copy(g_h.at[src_t.at[jnp.int32(0)]],
                              bbs[0], sem).wait()

    def _fire_gather(c, slot):
        pltpu.async_copy(g_h.at[src_t.at[c]], bbs[slot], gsems[slot])

    def _fire_scatter(c, slot, src):
        pltpu.async_copy(src, s_sh.at[dst_t.at[c]], ssems[slot], add=True)

    def _pipe(gather_src):
        hop = gather_src is None
        if hop:
            for x in range(L):
                _fire_gather(jnp.int32(x), x % U)

        def outer(i):
            for j in range(U):
                b = i * jnp.int32(U) + jnp.int32(j)
                sf = (j + L) % U

                @pl.when(jnp.logical_and(b + L < NCHT, b + L >= U))
                def _():
                    _drain(ssems[sf])

                if hop:
                    @pl.when(b + L < NCHT)
                    def _():
                        _fire_gather(b + L, sf)

                    _drain(gsems[j])
                    _fire_scatter(b, j, bbs[j])
                else:
                    _fire_scatter(b, j, gather_src)

        _loop(NCHT // U, outer)
        for j in range(U):
            _drain(ssems[j])

    def _publish():
        plsc.subcore_barrier()
        pltpu.sync_copy(s_sh.at[rows16], pub_h.at[cid].at[rows16])
        _xbar()

    def _refill_s():
        @pl.when(cid == jnp.int32(0))
        def _():
            pltpu.sync_copy(g_h.at[rows16], s_sh.at[rows16])

        @pl.when(cid == jnp.int32(1))
        def _():
            pltpu.sync_copy(zbuf, s_sh.at[rows16])

        plsc.subcore_barrier()

    pltpu.sync_copy(src_h.at[cid].at[sid], src_t)
    pltpu.sync_copy(dst_h.at[cid].at[sid], dst_t)

    zero_v = jnp.full((C,), 0.0, jnp.float32)
    ones_v = jnp.full((C,), 1.0, jnp.float32)

    def _fill(r):
        zbuf[r] = zero_v
        zbuf[r + jnp.int32(RPW)] = zero_v
        tbuf[r] = ones_v

    _loop_rows(_fill)
    pltpu.sync_copy(zbuf, s_sh.at[rows16])
    plsc.subcore_barrier()

    _pipe(gather_src=tbuf.at[pl.ds(0, CH)])
    _publish()

    pltpu.sync_copy(s_sh.at[rows32], tbuf)
    pltpu.sync_copy(pub_h.at[ocid].at[rows32], pbuf)
    pltpu.sync_copy(y_h.at[rows32], ybuf)

    def _dis_row(r):
        dv = _rsqrt16(tbuf[r] + pbuf[r] + 1.0)
        disb[r] = dv
        ybuf[r] = ybuf[r] * dv

    _loop_rows(_dis_row)
    pltpu.sync_copy(ybuf, g_h.at[rows32])
    _xbar()
    _refill_s()

    _pipe(gather_src=None)
    _publish()

    pltpu.sync_copy(s_sh.at[rows32], tbuf)
    pltpu.sync_copy(pub_h.at[ocid].at[rows32], pbuf)

    def _g1_row(r):
        dv = disb[r]
        ybuf[r] = (tbuf[r] + pbuf[r]) * dv * dv

    _loop_rows(_g1_row)
    pltpu.sync_copy(ybuf, g_h.at[rows32])
    _xbar()
    _refill_s()

    _pipe(gather_src=None)
    _publish()

    pltpu.sync_copy(s_sh.at[rows32], tbuf)
    pltpu.sync_copy(pub_h.at[ocid].at[rows32], pbuf)

    def _h2_row(r):
        ybuf[r] = (tbuf[r] + pbuf[r]) * disb[r]

    _loop_rows(_h2_row)
    pltpu.sync_copy(ybuf, h2_h.at[rows32])


_sc_prop = functools.partial(
    pl.kernel,
    out_type=[
        jax.ShapeDtypeStruct((NP, C), jnp.float32),
        jax.ShapeDtypeStruct((NP, C), jnp.float32),
        jax.ShapeDtypeStruct((NC, NP, C), jnp.float32),
    ],
    mesh=plsc.VectorSubcoreMesh(
        core_axis_name="c", subcore_axis_name="s", num_cores=NC),
    compiler_params=pltpu.CompilerParams(
        needs_layout_passes=False, use_tc_tiling_on_sc=False),
    scratch_types=[
        pltpu.VMEM_SHARED((NP, C), jnp.float32),
        pltpu.VMEM((NCHT, CH), jnp.int32),
        pltpu.VMEM((NCHT, CH), jnp.int32),
        [pltpu.VMEM((CH, C), jnp.float32) for _ in range(U)],
        pltpu.VMEM((RPW, C), jnp.float32),
        pltpu.VMEM((RPW, C), jnp.float32),
        pltpu.VMEM((RPW, C), jnp.float32),
        pltpu.VMEM((RPW, C), jnp.float32),
        pltpu.VMEM((RPS, C), jnp.float32),
        [pltpu.SemaphoreType.DMA for _ in range(U)],
        [pltpu.SemaphoreType.DMA for _ in range(U)],
        pltpu.SemaphoreType.REGULAR,
    ],
)(_sc_body)


def _matmul_body(x_ref, w_ref, o_ref):
    o_ref[...] = jnp.dot(x_ref[...], w_ref[...],
                         preferred_element_type=jnp.float32)


def _lsm_body(h_ref, b_ref, o_ref):
    t = h_ref[...] + b_ref[...]
    m = jnp.max(t, axis=1, keepdims=True)
    e = jnp.exp(t - m)
    s = jnp.sum(e, axis=1, keepdims=True)
    o_ref[...] = t - m - jnp.log(s)


def kernel(x, edge_index, W, b):
    out_dtype = jnp.result_type(x.dtype, W.dtype, b.dtype)
    x = x.astype(jnp.float32)
    W = W.astype(jnp.float32)
    b = b.astype(jnp.float32)
    ei = jnp.pad(edge_index, ((0, 0), (0, EP - edge_index.shape[1])),
                 constant_values=PAD_NODE).astype(jnp.int32)
    ei = ei.reshape(2, NC, NS, NCHT, CH)
    src = ei[0]
    dst = ei[1]
    xp = jnp.pad(x, ((0, NP - N), (0, 0)))

    y = pl.pallas_call(
        _matmul_body,
        out_shape=jax.ShapeDtypeStruct((NP, C), jnp.float32),
    )(xp, W)

    h2, _, _ = _sc_prop(src, dst, y)

    out = pl.pallas_call(
        _lsm_body,
        out_shape=jax.ShapeDtypeStruct((NP, C), jnp.float32),
    )(h2, b.reshape(1, C))

    return out[:N].astype(out_dtype)


# speedup vs baseline: 1.1282x; 1.0066x over previous
import functools

import jax
import jax.numpy as jnp
from jax import lax
from jax.experimental import pallas as pl
from jax.experimental.pallas import tpu as pltpu
from jax.experimental.pallas import tpu_sc as plsc

N = 10000
D = 128
C = 16
NC = 2
NS = 16
NW = NC * NS
RPW = 320
RPS = 640
NP = NW * RPW
CH = 128
E = 320000
U = 10
L = 5
NCHT = 80
EPT = NCHT * CH
EP = NW * EPT
PAD_NODE = NP - 1
UNR = 8


def _loop(n, body):
    lax.fori_loop(jnp.int32(0), jnp.int32(n), lambda i, c: (body(i), c)[1],
                  None)


def _loop_rows(body):
    def outer(i):
        base = i * jnp.int32(UNR)
        for k in range(UNR):
            body(base + jnp.int32(k))
    _loop(RPW // UNR, outer)


def _rsqrt16(d):
    i = plsc.bitcast(d, jnp.int32)
    i = jnp.int32(0x5F3759DF) - lax.shift_right_logical(i, jnp.int32(1))
    y = plsc.bitcast(i, jnp.float32)
    for _ in range(3):
        y = y * (1.5 - 0.5 * d * y * y)
    return y


def _sc_body(src_h, dst_h, y_h, h2_h, g_h, pub_h, s_sh, src_t, dst_t,
             bbs, tbuf, pbuf, ybuf, disb, zbuf, gsems, ssems, xsem):
    cid = lax.axis_index("c")
    sid = lax.axis_index("s")
    ocid = jnp.int32(1) - cid
    wid = cid * jnp.int32(NS) + sid
    rows32 = pl.ds(wid * jnp.int32(RPW), RPW)
    rows16 = pl.ds(sid * jnp.int32(RPS), RPS)

    def _xbar():
        plsc.subcore_barrier()

        @pl.when(sid == jnp.int32(0))
        def _():
            pl.semaphore_signal(xsem, 1, core_index=ocid)
            pl.semaphore_wait(xsem, 1)

        plsc.subcore_barrier()

    def _drain(sem):
        pltpu.make_async_copy(g_h.at[src_t.at[jnp.int32(0)]],
                              bbs[0], sem).wait()

    def _fire_gather(c, slot):
        pltpu.async_copy(g_h.at[src_t.at[c]], bbs[slot], gsems[slot])

    def _fire_scatter(c, slot, src):
        pltpu.async_copy(src, s_sh.at[dst_t.at[c]], ssems[slot], add=True)

    def _pipe(gather_src):
        hop = gather_src is None
        if hop:
            for x in range(L):
                _fire_gather(jnp.int32(x), x % U)

        def outer(i):
            for j in range(U):
                b = i * jnp.int32(U) + jnp.int32(j)
                sf = (j + L) % U

                @pl.when(jnp.logical_and(b + L < NCHT, b + L >= U))
                def _():
                    _drain(ssems[sf])

                if hop:
                    @pl.when(b + L < NCHT)
                    def _():
                        _fire_gather(b + L, sf)

                    _drain(gsems[j])
                    _fire_scatter(b, j, bbs[j])
                else:
                    _fire_scatter(b, j, gather_src)

        _loop(NCHT // U, outer)
        for j in range(U):
            _drain(ssems[j])

    def _publish():
        plsc.subcore_barrier()
        pltpu.sync_copy(s_sh.at[rows16], pub_h.at[cid].at[rows16])
        _xbar()

    def _refill_s():
        @pl.when(cid == jnp.int32(0))
        def _():
            pltpu.sync_copy(g_h.at[rows16], s_sh.at[rows16])

        @pl.when(cid == jnp.int32(1))
        def _():
            pltpu.sync_copy(zbuf, s_sh.at[rows16])

        plsc.subcore_barrier()

    pltpu.sync_copy(src_h.at[cid].at[sid], src_t)
    pltpu.sync_copy(dst_h.at[cid].at[sid], dst_t)

    zero_v = jnp.full((C,), 0.0, jnp.float32)
    ones_v = jnp.full((C,), 1.0, jnp.float32)

    def _fill(r):
        zbuf[r] = zero_v
        zbuf[r + jnp.int32(RPW)] = zero_v
        tbuf[r] = ones_v

    _loop_rows(_fill)
    pltpu.sync_copy(zbuf, s_sh.at[rows16])
    plsc.subcore_barrier()

    _pipe(gather_src=tbuf.at[pl.ds(0, CH)])
    _publish()

    d1 = pltpu.async_copy(s_sh.at[rows32], tbuf, gsems[0])
    d2 = pltpu.async_copy(pub_h.at[ocid].at[rows32], pbuf, gsems[1])
    d3 = pltpu.async_copy(y_h.at[rows32], ybuf, gsems[2])
    d1.wait()
    d2.wait()
    d3.wait()

    def _dis_row(r):
        dv = _rsqrt16(tbuf[r] + pbuf[r] + 1.0)
        disb[r] = dv
        ybuf[r] = ybuf[r] * dv

    _loop_rows(_dis_row)
    pltpu.sync_copy(ybuf, g_h.at[rows32])
    _xbar()
    _refill_s()

    _pipe(gather_src=None)
    _publish()

    d1 = pltpu.async_copy(s_sh.at[rows32], tbuf, gsems[0])
    d2 = pltpu.async_copy(pub_h.at[ocid].at[rows32], pbuf, gsems[1])
    d1.wait()
    d2.wait()

    def _g1_row(r):
        dv = disb[r]
        ybuf[r] = (tbuf[r] + pbuf[r]) * dv * dv

    _loop_rows(_g1_row)
    pltpu.sync_copy(ybuf, g_h.at[rows32])
    _xbar()
    _refill_s()

    _pipe(gather_src=None)
    _publish()

    d1 = pltpu.async_copy(s_sh.at[rows32], tbuf, gsems[0])
    d2 = pltpu.async_copy(pub_h.at[ocid].at[rows32], pbuf, gsems[1])
    d1.wait()
    d2.wait()

    def _h2_row(r):
        ybuf[r] = (tbuf[r] + pbuf[r]) * disb[r]

    _loop_rows(_h2_row)
    pltpu.sync_copy(ybuf, h2_h.at[rows32])


_sc_prop = functools.partial(
    pl.kernel,
    out_type=[
        jax.ShapeDtypeStruct((NP, C), jnp.float32),
        jax.ShapeDtypeStruct((NP, C), jnp.float32),
        jax.ShapeDtypeStruct((NC, NP, C), jnp.float32),
    ],
    mesh=plsc.VectorSubcoreMesh(
        core_axis_name="c", subcore_axis_name="s", num_cores=NC),
    compiler_params=pltpu.CompilerParams(
        needs_layout_passes=False, use_tc_tiling_on_sc=False),
    scratch_types=[
        pltpu.VMEM_SHARED((NP, C), jnp.float32),
        pltpu.VMEM((NCHT, CH), jnp.int32),
        pltpu.VMEM((NCHT, CH), jnp.int32),
        [pltpu.VMEM((CH, C), jnp.float32) for _ in range(U)],
        pltpu.VMEM((RPW, C), jnp.float32),
        pltpu.VMEM((RPW, C), jnp.float32),
        pltpu.VMEM((RPW, C), jnp.float32),
        pltpu.VMEM((RPW, C), jnp.float32),
        pltpu.VMEM((RPS, C), jnp.float32),
        [pltpu.SemaphoreType.DMA for _ in range(U)],
        [pltpu.SemaphoreType.DMA for _ in range(U)],
        pltpu.SemaphoreType.REGULAR,
    ],
)(_sc_body)


def _matmul_body(x_ref, w_ref, o_ref):
    o_ref[...] = jnp.dot(x_ref[...], w_ref[...],
                         preferred_element_type=jnp.float32)


def _lsm_body(h_ref, b_ref, o_ref):
    t = h_ref[...] + b_ref[...]
    m = jnp.max(t, axis=1, keepdims=True)
    e = jnp.exp(t - m)
    s = jnp.sum(e, axis=1, keepdims=True)
    o_ref[...] = t - m - jnp.log(s)


def kernel(x, edge_index, W, b):
    out_dtype = jnp.result_type(x.dtype, W.dtype, b.dtype)
    x = x.astype(jnp.float32)
    W = W.astype(jnp.float32)
    b = b.astype(jnp.float32)
    ei = jnp.pad(edge_index, ((0, 0), (0, EP - edge_index.shape[1])),
                 constant_values=PAD_NODE).astype(jnp.int32)
    ei = ei.reshape(2, NC, NS, NCHT, CH)
    src = ei[0]
    dst = ei[1]
    xp = jnp.pad(x, ((0, NP - N), (0, 0)))

    y = pl.pallas_call(
        _matmul_body,
        out_shape=jax.ShapeDtypeStruct((NP, C), jnp.float32),
    )(xp, W)

    h2, _, _ = _sc_prop(src, dst, y)

    out = pl.pallas_call(
        _lsm_body,
        out_shape=jax.ShapeDtypeStruct((NP, C), jnp.float32),
    )(h2, b.reshape(1, C))

    return out[:N].astype(out_dtype)
